# baseline (speedup 1.0000x reference)
import functools

import jax
import jax.numpy as jnp
from jax import lax
from jax.experimental import pallas as pl
from jax.experimental.pallas import tpu as pltpu
from jax.experimental.pallas import tpu_sc as plsc

N = 10000
D = 128
BN_EPS = 1e-5

NUM_CORES = 2
NUM_SUBCORES = 16
NUM_TILES = NUM_CORES * NUM_SUBCORES

W = 128
KW = 20
NB = 2
E_IN = 320000
STEP = KW * W
CPT = -(-E_IN // (STEP * NUM_TILES))
EPAD = NUM_TILES * CPT * STEP
NPAD = 10112
RPT = NPAD // NUM_SUBCORES


@jax.jit
def _sc_aggregate(x, ei):
    _vector_mesh = plsc.VectorSubcoreMesh(
        core_axis_name="core", subcore_axis_name="subcore",
        num_cores=NUM_CORES, num_subcores=NUM_SUBCORES)
    ei5 = ei.reshape(2, NUM_TILES, CPT, KW, W)

    @functools.partial(
        pl.kernel,
        out_type=jax.ShapeDtypeStruct((NUM_CORES, NPAD, D), jnp.float32),
        mesh=_vector_mesh,
        scratch_types=(
            [pltpu.VMEM_SHARED((NPAD, D), jnp.float32)]
            + [pltpu.VMEM((W, D), jnp.float32)] * NB
            + [pltpu.SemaphoreType.DMA] * (2 * NB)
        ),
    )
    def agg_kernel(x_hbm, ei_hbm, out_hbm, acc_spmem, *scratch):
        cid = lax.axis_index("core")
        sid = lax.axis_index("subcore")
        rows = scratch[:NB]
        gsem = scratch[NB:2 * NB]
        ssem = scratch[2 * NB:3 * NB]
        rows_a = rows[0]

        @pl.loop(0, W)
        def _(i):
            for r in rows:
                for g in range(D // 16):
                    r[pl.ds(i, 1), pl.ds(g * 16, 16)] = jnp.zeros(
                        (1, 16), jnp.float32)

        zbase = sid * RPT
        nfull = RPT // W
        for k in range(nfull):
            pltpu.sync_copy(rows[k % NB],
                            acc_spmem.at[pl.ds(zbase + k * W, W)])
        zrem = RPT % W
        if zrem:
            pltpu.sync_copy(rows_a.at[pl.ds(0, zrem)],
                            acc_spmem.at[pl.ds(zbase + nfull * W, zrem)])
        plsc.subcore_barrier()

        def window(src_idx, dst_idx):
            gd = [None] * KW
            sd = [None] * KW
            for j in range(NB - 1):
                gd[j] = pltpu.async_copy(
                    x_hbm.at[src_idx.at[0, 0, 0, j]], rows[j % NB],
                    gsem[j % NB])
            for j in range(KW):
                b = j % NB
                jn = j + NB - 1
                if jn < KW:
                    if j >= 1:
                        sd[j - 1].wait()
                    gd[jn] = pltpu.async_copy(
                        x_hbm.at[src_idx.at[0, 0, 0, jn]], rows[jn % NB],
                        gsem[jn % NB])
                gd[j].wait()
                sd[j] = pltpu.async_copy(
                    rows[b], acc_spmem.at[dst_idx.at[0, 0, 0, j]], ssem[b],
                    add=True)
            for j in range(KW - NB + 1, KW):
                if j >= 0 and sd[j] is not None:
                    sd[j].wait()
            sd[KW - NB].wait()

        pltpu.emit_pipeline(
            window,
            grid=(NUM_TILES, CPT),
            in_specs=[
                pl.BlockSpec((1, 1, 1, KW, W), lambda t, i: (0, t, i, 0, 0)),
                pl.BlockSpec((1, 1, 1, KW, W), lambda t, i: (1, t, i, 0, 0)),
            ],
            core_axis_name=("core", "subcore"),
            dimension_semantics=(pltpu.PARALLEL, pltpu.ARBITRARY),
        )(ei_hbm, ei_hbm)

        plsc.subcore_barrier()
        obase = sid * RPT
        pltpu.sync_copy(acc_spmem.at[pl.ds(obase, RPT)],
                        out_hbm.at[cid, pl.ds(obase, RPT)])

    return agg_kernel(x, ei5)


GB = 10
BR = N // GB


def _mlp_grid_body(x_ref, p_ref, w1t_ref, b1_ref, gamma_ref, beta_ref,
                   w2t_ref, b2_ref, o_ref, h1_ref, sums_ref, stats_ref):
    ph = pl.program_id(0)
    i = pl.program_id(1)

    @pl.when(ph == 0)
    def _():
        h = x_ref[...] + p_ref[0] + p_ref[1]
        h1 = jnp.dot(h, w1t_ref[...], preferred_element_type=jnp.float32)
        h1 = h1 + b1_ref[...]
        h1_ref[pl.ds(i * BR, BR), :] = h1
        s = jnp.sum(h1, axis=0, keepdims=True)
        sq = jnp.sum(h1 * h1, axis=0, keepdims=True)

        @pl.when(i == 0)
        def _():
            sums_ref[0:1, :] = s
            sums_ref[1:2, :] = sq

        @pl.when(i > 0)
        def _():
            sums_ref[0:1, :] = sums_ref[0:1, :] + s
            sums_ref[1:2, :] = sums_ref[1:2, :] + sq

        @pl.when(i == GB - 1)
        def _():
            mean = sums_ref[0:1, :] * (1.0 / N)
            var = sums_ref[1:2, :] * (1.0 / N) - mean * mean
            stats_ref[0:1, :] = mean
            stats_ref[1:2, :] = lax.rsqrt(var + BN_EPS)

    @pl.when(ph == 1)
    def _():
        h1 = h1_ref[pl.ds(i * BR, BR), :]
        hn = ((h1 - stats_ref[0:1, :]) * stats_ref[1:2, :] * gamma_ref[...]
              + beta_ref[...])
        h2 = jnp.maximum(hn, 0.0)
        o = jnp.dot(h2, w2t_ref[...], preferred_element_type=jnp.float32)
        o_ref[...] = jnp.maximum(o + b2_ref[...], 0.0)


@jax.jit
def _tc_mlp_grid(x, p, w1t, b1, gamma, beta, w2t, b2):
    full = lambda ph, i: (0, 0)
    return pl.pallas_call(
        _mlp_grid_body,
        grid=(2, GB),
        in_specs=[
            pl.BlockSpec((BR, D), lambda ph, i: (jnp.where(ph == 0, i, 0), 0)),
            pl.BlockSpec((2, BR, D),
                         lambda ph, i: (0, jnp.where(ph == 0, i, 0), 0)),
            pl.BlockSpec((D, D), full),
            pl.BlockSpec((1, D), full),
            pl.BlockSpec((1, D), full),
            pl.BlockSpec((1, D), full),
            pl.BlockSpec((D, D), full),
            pl.BlockSpec((1, D), full),
        ],
        out_specs=pl.BlockSpec((BR, D),
                               lambda ph, i: (jnp.where(ph == 1, i, 0), 0)),
        out_shape=jax.ShapeDtypeStruct((N, D), jnp.float32),
        scratch_shapes=[
            pltpu.VMEM((N, D), jnp.float32),
            pltpu.VMEM((2, D), jnp.float32),
            pltpu.VMEM((2, D), jnp.float32),
        ],
        compiler_params=pltpu.CompilerParams(
            vmem_limit_bytes=100 * 1024 * 1024,
            dimension_semantics=("arbitrary", "arbitrary")),
    )(x, p, w1t, b1.reshape(1, D), gamma.reshape(1, D), beta.reshape(1, D),
      w2t, b2.reshape(1, D))


def _mlp_body(x_ref, p_ref, w1t_ref, b1_ref, gamma_ref, beta_ref, w2t_ref,
              b2_ref, o_ref):
    h = x_ref[...] + p_ref[0, pl.ds(0, N)] + p_ref[1, pl.ds(0, N)]
    h1 = jnp.dot(h, w1t_ref[...], preferred_element_type=jnp.float32)
    h1 = h1 + b1_ref[...]
    mean = jnp.mean(h1, axis=0, keepdims=True)
    c = h1 - mean
    var = jnp.mean(c * c, axis=0, keepdims=True)
    hn = c * lax.rsqrt(var + BN_EPS) * gamma_ref[...] + beta_ref[...]
    h2 = jnp.maximum(hn, 0.0)
    o = jnp.dot(h2, w2t_ref[...], preferred_element_type=jnp.float32)
    o = o + b2_ref[...]
    o_ref[...] = jnp.maximum(o, 0.0)


@jax.jit
def _tc_mlp(x, p, w1t, b1, gamma, beta, w2t, b2):
    return pl.pallas_call(
        _mlp_body,
        out_shape=jax.ShapeDtypeStruct((N, D), jnp.float32),
        compiler_params=pltpu.CompilerParams(
            vmem_limit_bytes=100 * 1024 * 1024),
    )(x, p, w1t, b1.reshape(1, D), gamma.reshape(1, D), beta.reshape(1, D),
      w2t, b2.reshape(1, D))


def kernel(x, edge_index, W1, b1, gamma, beta, W2, b2):
    npad = EPAD - E_IN
    pad_ids = jnp.arange(npad, dtype=jnp.int32)
    pads = jnp.stack([(pad_ids * 37) % N, N + pad_ids % (NPAD - N)])
    ei_full = jnp.concatenate([edge_index.astype(jnp.int32), pads], axis=1)
    p = _sc_aggregate(x, ei_full)
    return _tc_mlp_grid(x, p, W1.T, b1, gamma, beta, W2.T, b2)


# speedup vs baseline: 1.1440x; 1.1440x over previous
import functools

import jax
import jax.numpy as jnp
from jax import lax
from jax.experimental import pallas as pl
from jax.experimental.pallas import tpu as pltpu
from jax.experimental.pallas import tpu_sc as plsc

N = 10000
D = 128
BN_EPS = 1e-5

NUM_CORES = 2
NUM_SUBCORES = 16
NUM_TILES = NUM_CORES * NUM_SUBCORES

W = 128
KW = 20
NB = 2
E_IN = 320000
STEP = KW * W
CPT = -(-E_IN // (STEP * NUM_TILES))
EPAD = NUM_TILES * CPT * STEP
NPAD = 10112
RPT = NPAD // NUM_SUBCORES


@jax.jit
def _sc_aggregate(x, ei):
    _vector_mesh = plsc.VectorSubcoreMesh(
        core_axis_name="core", subcore_axis_name="subcore",
        num_cores=NUM_CORES, num_subcores=NUM_SUBCORES)
    ei5 = ei.reshape(2, NUM_TILES, CPT, KW, W)

    @functools.partial(
        pl.kernel,
        out_type=jax.ShapeDtypeStruct((NUM_CORES, NPAD, D), jnp.float32),
        mesh=_vector_mesh,
        scratch_types=(
            [pltpu.VMEM_SHARED((NPAD, D), jnp.float32)]
            + [pltpu.VMEM((W, D), jnp.float32)] * NB
            + [pltpu.SemaphoreType.DMA] * (2 * NB)
        ),
    )
    def agg_kernel(x_hbm, ei_hbm, out_hbm, acc_spmem, *scratch):
        cid = lax.axis_index("core")
        sid = lax.axis_index("subcore")
        rows = scratch[:NB]
        gsem = scratch[NB:2 * NB]
        ssem = scratch[2 * NB:3 * NB]
        rows_a = rows[0]

        @pl.loop(0, W)
        def _(i):
            for r in rows:
                for g in range(D // 16):
                    r[pl.ds(i, 1), pl.ds(g * 16, 16)] = jnp.zeros(
                        (1, 16), jnp.float32)

        zbase = sid * RPT
        nfull = RPT // W
        for k in range(nfull):
            pltpu.sync_copy(rows[k % NB],
                            acc_spmem.at[pl.ds(zbase + k * W, W)])
        zrem = RPT % W
        if zrem:
            pltpu.sync_copy(rows_a.at[pl.ds(0, zrem)],
                            acc_spmem.at[pl.ds(zbase + nfull * W, zrem)])
        plsc.subcore_barrier()

        def window(src_idx, dst_idx):
            gd = [None] * KW
            sd = [None] * KW
            for j in range(NB - 1):
                gd[j] = pltpu.async_copy(
                    x_hbm.at[src_idx.at[0, 0, 0, j]], rows[j % NB],
                    gsem[j % NB])
            for j in range(KW):
                b = j % NB
                jn = j + NB - 1
                if jn < KW:
                    if j >= 1:
                        sd[j - 1].wait()
                    gd[jn] = pltpu.async_copy(
                        x_hbm.at[src_idx.at[0, 0, 0, jn]], rows[jn % NB],
                        gsem[jn % NB])
                gd[j].wait()
                sd[j] = pltpu.async_copy(
                    rows[b], acc_spmem.at[dst_idx.at[0, 0, 0, j]], ssem[b],
                    add=True)
            for j in range(KW - NB + 1, KW):
                if j >= 0 and sd[j] is not None:
                    sd[j].wait()
            sd[KW - NB].wait()

        pltpu.emit_pipeline(
            window,
            grid=(NUM_TILES, CPT),
            in_specs=[
                pl.BlockSpec((1, 1, 1, KW, W), lambda t, i: (0, t, i, 0, 0)),
                pl.BlockSpec((1, 1, 1, KW, W), lambda t, i: (1, t, i, 0, 0)),
            ],
            core_axis_name=("core", "subcore"),
            dimension_semantics=(pltpu.PARALLEL, pltpu.ARBITRARY),
        )(ei_hbm, ei_hbm)

        plsc.subcore_barrier()
        obase = sid * RPT
        pltpu.sync_copy(acc_spmem.at[pl.ds(obase, RPT)],
                        out_hbm.at[cid, pl.ds(obase, RPT)])

    return agg_kernel(x, ei5)


def _mlp_body(x_ref, p_ref, w1t_ref, b1_ref, gamma_ref, beta_ref, w2t_ref,
              b2_ref, o_ref):
    h = x_ref[...] + p_ref[0, pl.ds(0, N)] + p_ref[1, pl.ds(0, N)]
    h1 = jnp.dot(h, w1t_ref[...], preferred_element_type=jnp.float32)
    h1 = h1 + b1_ref[...]
    mean = jnp.mean(h1, axis=0, keepdims=True)
    c = h1 - mean
    var = jnp.mean(c * c, axis=0, keepdims=True)
    hn = c * lax.rsqrt(var + BN_EPS) * gamma_ref[...] + beta_ref[...]
    h2 = jnp.maximum(hn, 0.0)
    o = jnp.dot(h2, w2t_ref[...], preferred_element_type=jnp.float32)
    o = o + b2_ref[...]
    o_ref[...] = jnp.maximum(o, 0.0)


@jax.jit
def _tc_mlp(x, p, w1t, b1, gamma, beta, w2t, b2):
    return pl.pallas_call(
        _mlp_body,
        out_shape=jax.ShapeDtypeStruct((N, D), jnp.float32),
        compiler_params=pltpu.CompilerParams(
            vmem_limit_bytes=100 * 1024 * 1024),
    )(x, p, w1t, b1.reshape(1, D), gamma.reshape(1, D), beta.reshape(1, D),
      w2t, b2.reshape(1, D))


def kernel(x, edge_index, W1, b1, gamma, beta, W2, b2):
    npad = EPAD - E_IN
    pad_ids = jnp.arange(npad, dtype=jnp.int32)
    pads = jnp.stack([(pad_ids * 37) % N, N + pad_ids % (NPAD - N)])
    ei_full = jnp.concatenate([edge_index.astype(jnp.int32), pads], axis=1)
    p = _sc_aggregate(x, ei_full)
    return _tc_mlp(x, p, W1.T, b1, gamma, beta, W2.T, b2)
